# baseline (speedup 1.0000x reference)
import functools

import jax
import jax.numpy as jnp
from jax import lax
from jax.experimental import pallas as pl
from jax.experimental.pallas import tpu as pltpu
from jax.experimental.pallas import tpu_sc as plsc

V = 1_000_000
K = 8
L = 16
NTILES = 16
NW = 32

SCNT = 31_264
SCNT_L = V - (NW - 1) * SCNT
SNV = SCNT // L
SNVL = SCNT_L // L
SG = 16
SNSG = SNV // SG
STAILV = SNV - SNSG * SG

CNT = 62_528
CNT_L = V - (NTILES - 1) * CNT
ZB = 15_632
NZ = CNT // ZB
NZL_FULL = CNT_L // ZB
ZREM = CNT_L - NZL_FULL * ZB

CB = 1024
NEG = float("-inf")
BIGI = 2**30

_DNUMS = lax.GatherDimensionNumbers(
    offset_dims=(), collapsed_slice_dims=(0,), start_index_map=(0,))

_mesh = plsc.VectorSubcoreMesh(
    core_axis_name="c", subcore_axis_name="s", num_cores=2)

_rmesh = plsc.VectorSubcoreMesh(
    core_axis_name="rc", subcore_axis_name="rs", num_cores=2)


def _g16(x, idx):
  return lax.gather(x, idx.reshape(L, 1), _DNUMS, (1,),
                    mode=lax.GatherScatterMode.PROMISE_IN_BOUNDS)


def _bf(x, iota, op):
  for k in range(4):
    x = op(x, _g16(x, jnp.bitwise_xor(iota, 1 << k)))
  return x


def _select8(read_v, read_i, nvecs, iota):
  negv = jnp.full((L,), NEG, jnp.float32)
  bigv = jnp.full((L,), BIGI, jnp.int32)
  outv = negv
  outi = bigv
  pvv = jnp.full((L,), float("inf"), jnp.float32)
  piv = jnp.full((L,), -1, jnp.int32)
  for r in range(K):
    def scan(k, carry, pvv=pvv, piv=piv):
      bv, bi = carry
      v = read_v(k)
      ix = read_i(k)
      elig = (v < pvv) | ((v == pvv) & (ix > piv))
      vv = jnp.where(elig, v, negv)
      better = (vv > bv) | ((vv == bv) & (ix < bi))
      bv = jnp.where(better, vv, bv)
      bi = jnp.where(better, ix, bi)
      return bv, bi

    bv, bi = lax.fori_loop(0, nvecs, scan, (negv, bigv))
    mvv = _bf(bv, iota, jnp.maximum)
    miv = _bf(jnp.where(bv == mvv, bi, bigv), iota, jnp.minimum)
    outv = jnp.where(iota == r, mvv, outv)
    outi = jnp.where(iota == r, miv, outi)
    pvv, piv = mvv, miv
  return outv, outi


def _zero_fill(out, zbuf, sem, zbase, wid, last):
  @pl.when(wid < last)
  def _():
    for q in range(NZ):
      pltpu.async_copy(zbuf, out.at[pl.ds(zbase + q * ZB, ZB)], sem)

  @pl.when(wid == last)
  def _():
    for q in range(NZL_FULL):
      pltpu.async_copy(zbuf, out.at[pl.ds(zbase + q * ZB, ZB)], sem)
    zrem_base = zbase + NZL_FULL * ZB
    pltpu.async_copy(zbuf.at[pl.ds(0, ZREM)],
                     out.at[pl.ds(zrem_base, ZREM)], sem)


def _zero_wait(out, zbuf, sem, zbase, wid, last):
  @pl.when(wid < last)
  def _():
    for q in range(NZ):
      pltpu.make_async_copy(
          zbuf, out.at[pl.ds(zbase + q * ZB, ZB)], sem).wait()

  @pl.when(wid == last)
  def _():
    for q in range(NZL_FULL):
      pltpu.make_async_copy(
          zbuf, out.at[pl.ds(zbase + q * ZB, ZB)], sem).wait()
    zrem_base = zbase + NZL_FULL * ZB
    pltpu.make_async_copy(zbuf.at[pl.ds(0, ZREM)],
                          out.at[pl.ds(zrem_base, ZREM)], sem).wait()


@functools.partial(
    pl.kernel,
    out_type=(jax.ShapeDtypeStruct((V,), jnp.float32),
              jax.ShapeDtypeStruct((V,), jnp.float32),
              jax.ShapeDtypeStruct((L,), jnp.float32),
              jax.ShapeDtypeStruct((L,), jnp.int32)),
    mesh=_mesh,
    scratch_types=[
        pltpu.VMEM((SCNT,), jnp.float32),
        pltpu.VMEM((ZB,), jnp.float32),
        pltpu.VMEM((SNSG * L,), jnp.float32),
        pltpu.VMEM((CB,), jnp.float32),
        pltpu.VMEM((CB,), jnp.int32),
        pltpu.VMEM((L,), jnp.float32),
        pltpu.VMEM((L,), jnp.int32),
        pltpu.VMEM_SHARED(((NTILES + 1) * L,), jnp.float32),
        pltpu.VMEM_SHARED(((NTILES + 1) * L,), jnp.int32),
        pltpu.VMEM(((NTILES + 1) * L,), jnp.float32),
        pltpu.VMEM(((NTILES + 1) * L,), jnp.int32),
        pltpu.VMEM((L,), jnp.float32),
        pltpu.VMEM((L,), jnp.int32),
        pltpu.VMEM((L,), jnp.float32),
        pltpu.VMEM((L,), jnp.int32),
        pltpu.SemaphoreType.DMA,
        pltpu.SemaphoreType.DMA,
        pltpu.SemaphoreType.DMA,
        pltpu.SemaphoreType.REGULAR @ _rmesh,
    ],
)
def _topk_multihot(logits, pert, khot, cvx, cix,
                   xbuf, zbuf, gbuf, cv, ci, tv, ti, sh_v, sh_i, lvb, lib,
                   ones, gidx, rvb, rib, sem_in, sem_z0, sem_sc, rsem):
  cid = lax.axis_index("c")
  wid = lax.axis_index("s")
  w = cid * NTILES + wid
  sbase = w * SCNT
  zbase = wid * CNT
  iota = lax.iota(jnp.int32, L)
  last = NTILES - 1
  wlast = NW - 1
  negv = jnp.full((L,), NEG, jnp.float32)
  bigv = jnp.full((L,), BIGI, jnp.int32)
  zero = jnp.zeros((L,), jnp.float32)

  @pl.when(w < wlast)
  def _():
    pltpu.async_copy(logits.at[pl.ds(sbase, SCNT)], xbuf, sem_in)

  @pl.when(w == wlast)
  def _():
    pltpu.async_copy(logits.at[pl.ds(sbase, SCNT_L)],
                     xbuf.at[pl.ds(0, SCNT_L)], sem_in)

  def memset_body(i, _):
    o = i * (4 * L)
    zbuf[pl.ds(o, L)] = zero
    zbuf[pl.ds(o + L, L)] = zero
    zbuf[pl.ds(o + 2 * L, L)] = zero
    zbuf[pl.ds(o + 3 * L, L)] = zero
    return 0

  lax.fori_loop(0, ZB // (4 * L), memset_body, 0)
  zbuf[pl.ds(ZB - L, L)] = zero

  @pl.when(cid == 0)
  def _():
    _zero_fill(pert, zbuf, sem_z0, zbase, wid, last)

  @pl.when(cid == 1)
  def _():
    _zero_fill(khot, zbuf, sem_z0, zbase, wid, last)

  @pl.when(w < wlast)
  def _():
    pltpu.make_async_copy(logits.at[pl.ds(sbase, SCNT)], xbuf, sem_in).wait()

  @pl.when(w == wlast)
  def _():
    pltpu.make_async_copy(logits.at[pl.ds(sbase, SCNT_L)],
                          xbuf.at[pl.ds(0, SCNT_L)], sem_in).wait()
    for j in range(SNV - SNVL):
      xbuf[pl.ds(SCNT_L + j * L, L)] = negv

  def p1(sg, macc):
    o = sg * (SG * L)
    root = jnp.maximum(xbuf[pl.ds(o, L)], xbuf[pl.ds(o + L, L)])
    for j in range(2, SG):
      root = jnp.maximum(root, xbuf[pl.ds(o + j * L, L)])
    gbuf[pl.ds(sg * L, L)] = root
    return jnp.maximum(macc, root)

  macc = lax.fori_loop(0, SNSG, p1, negv)
  tail_o = SNSG * SG * L
  troot = xbuf[pl.ds(tail_o, L)]
  for j in range(1, STAILV):
    troot = jnp.maximum(troot, xbuf[pl.ds(tail_o + j * L, L)])
  macc = jnp.maximum(macc, troot)

  x = macc
  thrv = negv
  for r in range(K):
    thrv = _bf(x, iota, jnp.maximum)
    if r < K - 1:
      x = jnp.where(x == thrv, negv, x)
  thr_s = thrv[0]

  def vec_update(o, c):
    v = xbuf[pl.ds(o, L)]
    mask = v >= thrv
    vv = jnp.where(mask, v, negv)
    ii = jnp.where(mask, iota + (sbase + o), bigv)
    cv[pl.ds(c, L)] = vv
    ci[pl.ds(c, L)] = ii
    hit = _bf(vv, iota, jnp.maximum)[0] >= thr_s
    return c + jnp.where(hit, jnp.int32(L), jnp.int32(0))

  def p2(sg, c):
    root = gbuf[pl.ds(sg * L, L)]
    has = _bf(root, iota, jnp.maximum)[0] >= thr_s

    def upd(c):
      c = jnp.minimum(c, CB - SG * L)
      o = sg * (SG * L)
      for j in range(SG):
        c = vec_update(o + j * L, c)
      return c

    return lax.cond(has, upd, lambda c: c, c)

  c = lax.fori_loop(0, SNSG, p2, jnp.int32(0))
  c = jnp.minimum(c, CB - STAILV * L)
  for j in range(STAILV):
    c = vec_update(tail_o + j * L, c)

  outv, outi = _select8(lambda k: cv[pl.ds(k * L, L)],
                        lambda k: ci[pl.ds(k * L, L)], c // L, iota)
  tv[...] = outv
  ti[...] = outi
  pltpu.sync_copy(tv, sh_v.at[pl.ds(wid * L, L)])
  pltpu.sync_copy(ti, sh_i.at[pl.ds(wid * L, L)])

  @pl.when(cid == 0)
  def _():
    _zero_wait(pert, zbuf, sem_z0, zbase, wid, last)

  @pl.when(cid == 1)
  def _():
    _zero_wait(khot, zbuf, sem_z0, zbase, wid, last)

  plsc.subcore_barrier()

  target = {"rc": 0, "rs": 0}

  @pl.when(jnp.logical_and(cid == 1, wid == 0))
  def _():
    pltpu.sync_copy(sh_v.at[pl.ds(0, NTILES * L)], lvb.at[pl.ds(0, NTILES * L)])
    pltpu.sync_copy(sh_i.at[pl.ds(0, NTILES * L)], lib.at[pl.ds(0, NTILES * L)])
    v8, i8 = _select8(lambda k: lvb[pl.ds(k * L, L)],
                      lambda k: lib[pl.ds(k * L, L)], NTILES, iota)
    tv[...] = v8
    ti[...] = i8
    pltpu.sync_copy(tv, cvx)
    pltpu.sync_copy(ti, cix)
    pl.semaphore_signal(rsem, 1, device_id=target)

  @pl.when(jnp.logical_and(cid == 0, wid == 0))
  def _():
    pl.semaphore_wait(rsem, 1)
    pltpu.sync_copy(sh_v.at[pl.ds(0, NTILES * L)], lvb.at[pl.ds(0, NTILES * L)])
    pltpu.sync_copy(sh_i.at[pl.ds(0, NTILES * L)], lib.at[pl.ds(0, NTILES * L)])
    pltpu.sync_copy(cvx, lvb.at[pl.ds(NTILES * L, L)])
    pltpu.sync_copy(cix, lib.at[pl.ds(NTILES * L, L)])
    gv, gi = _select8(lambda k: lvb[pl.ds(k * L, L)],
                      lambda k: lib[pl.ds(k * L, L)], NTILES + 1, iota)
    del gv
    g0v = _g16(gi, jnp.zeros((L,), jnp.int32))
    gidx[...] = jnp.where(iota < K, gi, g0v)
    ones[...] = jnp.full((L,), 1.0, jnp.float32)
    pltpu.async_copy(ones, pert.at[gidx], sem_sc).wait()


def kernel(logits):
  pert, khot, _, _ = _topk_multihot(logits)
  return pert, khot


# speedup vs baseline: 1.0113x; 1.0113x over previous
import functools

import jax
import jax.numpy as jnp
from jax import lax
from jax.experimental import pallas as pl
from jax.experimental.pallas import tpu as pltpu
from jax.experimental.pallas import tpu_sc as plsc

V = 1_000_000
K = 8
L = 16
NTILES = 16
NW = 32

SCNT = 31_264
SCNT_L = V - (NW - 1) * SCNT
SNV = SCNT // L
SNVL = SCNT_L // L
SG = 16
SNSG = SNV // SG
STAILV = SNV - SNSG * SG

CNT = 62_528
CNT_L = V - (NTILES - 1) * CNT
ZB = 15_632
NZ = CNT // ZB
NZL_FULL = CNT_L // ZB
ZREM = CNT_L - NZL_FULL * ZB

CB = 1024
NEG = float("-inf")
BIGI = 2**30

_DNUMS = lax.GatherDimensionNumbers(
    offset_dims=(), collapsed_slice_dims=(0,), start_index_map=(0,))

_mesh = plsc.VectorSubcoreMesh(
    core_axis_name="c", subcore_axis_name="s", num_cores=2)

_rmesh = plsc.VectorSubcoreMesh(
    core_axis_name="rc", subcore_axis_name="rs", num_cores=2)


def _g16(x, idx):
  return lax.gather(x, idx.reshape(L, 1), _DNUMS, (1,),
                    mode=lax.GatherScatterMode.PROMISE_IN_BOUNDS)


def _bf(x, iota, op):
  for k in range(4):
    x = op(x, _g16(x, jnp.bitwise_xor(iota, 1 << k)))
  return x


def _select8(read_v, read_i, nvecs, iota):
  negv = jnp.full((L,), NEG, jnp.float32)
  bigv = jnp.full((L,), BIGI, jnp.int32)
  outv = negv
  outi = bigv
  pvv = jnp.full((L,), float("inf"), jnp.float32)
  piv = jnp.full((L,), -1, jnp.int32)
  for r in range(K):
    def scan(k, carry, pvv=pvv, piv=piv):
      bv, bi = carry
      v = read_v(k)
      ix = read_i(k)
      elig = (v < pvv) | ((v == pvv) & (ix > piv))
      vv = jnp.where(elig, v, negv)
      better = (vv > bv) | ((vv == bv) & (ix < bi))
      bv = jnp.where(better, vv, bv)
      bi = jnp.where(better, ix, bi)
      return bv, bi

    bv, bi = lax.fori_loop(0, nvecs, scan, (negv, bigv))
    mvv = _bf(bv, iota, jnp.maximum)
    miv = _bf(jnp.where(bv == mvv, bi, bigv), iota, jnp.minimum)
    outv = jnp.where(iota == r, mvv, outv)
    outi = jnp.where(iota == r, miv, outi)
    pvv, piv = mvv, miv
  return outv, outi


def _zero_fill(out, zbuf, sem, zbase, wid, last):
  @pl.when(wid < last)
  def _():
    for q in range(NZ):
      pltpu.async_copy(zbuf, out.at[pl.ds(zbase + q * ZB, ZB)], sem)

  @pl.when(wid == last)
  def _():
    for q in range(NZL_FULL):
      pltpu.async_copy(zbuf, out.at[pl.ds(zbase + q * ZB, ZB)], sem)
    zrem_base = zbase + NZL_FULL * ZB
    pltpu.async_copy(zbuf.at[pl.ds(0, ZREM)],
                     out.at[pl.ds(zrem_base, ZREM)], sem)


def _zero_wait(out, zbuf, sem, zbase, wid, last):
  @pl.when(wid < last)
  def _():
    for q in range(NZ):
      pltpu.make_async_copy(
          zbuf, out.at[pl.ds(zbase + q * ZB, ZB)], sem).wait()

  @pl.when(wid == last)
  def _():
    for q in range(NZL_FULL):
      pltpu.make_async_copy(
          zbuf, out.at[pl.ds(zbase + q * ZB, ZB)], sem).wait()
    zrem_base = zbase + NZL_FULL * ZB
    pltpu.make_async_copy(zbuf.at[pl.ds(0, ZREM)],
                          out.at[pl.ds(zrem_base, ZREM)], sem).wait()


@functools.partial(
    pl.kernel,
    out_type=(jax.ShapeDtypeStruct((V,), jnp.float32),
              jax.ShapeDtypeStruct((V,), jnp.float32),
              jax.ShapeDtypeStruct((L,), jnp.float32),
              jax.ShapeDtypeStruct((L,), jnp.int32)),
    mesh=_mesh,
    scratch_types=[
        pltpu.VMEM((SCNT,), jnp.float32),
        pltpu.VMEM((ZB,), jnp.float32),
        pltpu.VMEM((SNSG * L,), jnp.float32),
        pltpu.VMEM((CB,), jnp.float32),
        pltpu.VMEM((CB,), jnp.int32),
        pltpu.VMEM((L,), jnp.float32),
        pltpu.VMEM((L,), jnp.int32),
        pltpu.VMEM_SHARED(((NTILES + 1) * L,), jnp.float32),
        pltpu.VMEM_SHARED(((NTILES + 1) * L,), jnp.int32),
        pltpu.VMEM(((NTILES + 1) * L,), jnp.float32),
        pltpu.VMEM(((NTILES + 1) * L,), jnp.int32),
        pltpu.VMEM((L,), jnp.float32),
        pltpu.VMEM((L,), jnp.int32),
        pltpu.VMEM((L,), jnp.float32),
        pltpu.VMEM((L,), jnp.int32),
        pltpu.SemaphoreType.DMA,
        pltpu.SemaphoreType.DMA,
        pltpu.SemaphoreType.DMA,
        pltpu.SemaphoreType.REGULAR @ _rmesh,
    ],
)
def _topk_multihot(logits, pert, khot, cvx, cix,
                   xbuf, zbuf, gbuf, cv, ci, tv, ti, sh_v, sh_i, lvb, lib,
                   ones, gidx, rvb, rib, sem_in, sem_z0, sem_sc, rsem):
  cid = lax.axis_index("c")
  wid = lax.axis_index("s")
  w = cid * NTILES + wid
  sbase = w * SCNT
  zbase = wid * CNT
  iota = lax.iota(jnp.int32, L)
  last = NTILES - 1
  wlast = NW - 1
  negv = jnp.full((L,), NEG, jnp.float32)
  bigv = jnp.full((L,), BIGI, jnp.int32)
  zero = jnp.zeros((L,), jnp.float32)

  @pl.when(w < wlast)
  def _():
    pltpu.async_copy(logits.at[pl.ds(sbase, SCNT)], xbuf, sem_in)

  @pl.when(w == wlast)
  def _():
    pltpu.async_copy(logits.at[pl.ds(sbase, SCNT_L)],
                     xbuf.at[pl.ds(0, SCNT_L)], sem_in)

  def memset_body(i, _):
    o = i * (4 * L)
    zbuf[pl.ds(o, L)] = zero
    zbuf[pl.ds(o + L, L)] = zero
    zbuf[pl.ds(o + 2 * L, L)] = zero
    zbuf[pl.ds(o + 3 * L, L)] = zero
    return 0

  lax.fori_loop(0, ZB // (4 * L), memset_body, 0)
  zbuf[pl.ds(ZB - L, L)] = zero

  @pl.when(cid == 0)
  def _():
    _zero_fill(pert, zbuf, sem_z0, zbase, wid, last)

  @pl.when(cid == 1)
  def _():
    _zero_fill(khot, zbuf, sem_z0, zbase, wid, last)

  @pl.when(w < wlast)
  def _():
    pltpu.make_async_copy(logits.at[pl.ds(sbase, SCNT)], xbuf, sem_in).wait()

  @pl.when(w == wlast)
  def _():
    pltpu.make_async_copy(logits.at[pl.ds(sbase, SCNT_L)],
                          xbuf.at[pl.ds(0, SCNT_L)], sem_in).wait()
    for j in range(SNV - SNVL):
      xbuf[pl.ds(SCNT_L + j * L, L)] = negv

  def p1(sg, macc):
    o = sg * (SG * L)
    root = jnp.maximum(xbuf[pl.ds(o, L)], xbuf[pl.ds(o + L, L)])
    for j in range(2, SG):
      root = jnp.maximum(root, xbuf[pl.ds(o + j * L, L)])
    gbuf[pl.ds(sg * L, L)] = root
    return jnp.maximum(macc, root)

  macc = lax.fori_loop(0, SNSG, p1, negv)
  tail_o = SNSG * SG * L
  troot = xbuf[pl.ds(tail_o, L)]
  for j in range(1, STAILV):
    troot = jnp.maximum(troot, xbuf[pl.ds(tail_o + j * L, L)])
  macc = jnp.maximum(macc, troot)

  x = macc
  thrv = negv
  for r in range(K):
    thrv = _bf(x, iota, jnp.maximum)
    if r < K - 1:
      x = jnp.where(x == thrv, negv, x)
  thr_s = thrv[0]

  def vec_update(o, c):
    v = xbuf[pl.ds(o, L)]
    mask = v >= thrv
    vv = jnp.where(mask, v, negv)
    ii = jnp.where(mask, iota + (sbase + o), bigv)
    cv[pl.ds(c, L)] = vv
    ci[pl.ds(c, L)] = ii
    hit = _bf(vv, iota, jnp.maximum)[0] >= thr_s
    return c + jnp.where(hit, jnp.int32(L), jnp.int32(0))

  def p2(sg, c):
    root = gbuf[pl.ds(sg * L, L)]
    has = _bf(root, iota, jnp.maximum)[0] >= thr_s

    def upd(c):
      c = jnp.minimum(c, CB - SG * L)
      o = sg * (SG * L)
      for j in range(SG):
        c = vec_update(o + j * L, c)
      return c

    return lax.cond(has, upd, lambda c: c, c)

  c = lax.fori_loop(0, SNSG, p2, jnp.int32(0))
  c = jnp.minimum(c, CB - STAILV * L)
  for j in range(STAILV):
    c = vec_update(tail_o + j * L, c)

  outv, outi = _select8(lambda k: cv[pl.ds(k * L, L)],
                        lambda k: ci[pl.ds(k * L, L)], c // L, iota)
  tv[...] = outv
  ti[...] = outi
  pltpu.sync_copy(tv, sh_v.at[pl.ds(wid * L, L)])
  pltpu.sync_copy(ti, sh_i.at[pl.ds(wid * L, L)])

  @pl.when(cid == 0)
  def _():
    _zero_wait(pert, zbuf, sem_z0, zbase, wid, last)

  @pl.when(cid == 1)
  def _():
    _zero_wait(khot, zbuf, sem_z0, zbase, wid, last)

  plsc.subcore_barrier()

  target = {"rc": 0, "rs": 0}

  @pl.when(jnp.logical_and(cid == 1, wid == 0))
  def _():
    pltpu.sync_copy(sh_v.at[pl.ds(0, NTILES * L)], lvb.at[pl.ds(0, NTILES * L)])
    pltpu.sync_copy(sh_i.at[pl.ds(0, NTILES * L)], lib.at[pl.ds(0, NTILES * L)])
    v8, i8 = _select8(lambda k: lvb[pl.ds(k * L, L)],
                      lambda k: lib[pl.ds(k * L, L)], NTILES, iota)
    tv[...] = v8
    ti[...] = i8
    pltpu.sync_copy(tv, cvx)
    pltpu.sync_copy(ti, cix)
    pl.semaphore_signal(rsem, 1, device_id=target)

  @pl.when(jnp.logical_and(cid == 0, wid == 0))
  def _():
    d0 = pltpu.async_copy(
        sh_v.at[pl.ds(0, NTILES * L)], lvb.at[pl.ds(0, NTILES * L)], sem_in)
    d1 = pltpu.async_copy(
        sh_i.at[pl.ds(0, NTILES * L)], lib.at[pl.ds(0, NTILES * L)], sem_in)
    pl.semaphore_wait(rsem, 1)
    d2 = pltpu.async_copy(cvx, lvb.at[pl.ds(NTILES * L, L)], sem_z0)
    d3 = pltpu.async_copy(cix, lib.at[pl.ds(NTILES * L, L)], sem_z0)
    d0.wait()
    d1.wait()
    d2.wait()
    d3.wait()
    gv, gi = _select8(lambda k: lvb[pl.ds(k * L, L)],
                      lambda k: lib[pl.ds(k * L, L)], NTILES + 1, iota)
    del gv
    g0v = _g16(gi, jnp.zeros((L,), jnp.int32))
    gidx[...] = jnp.where(iota < K, gi, g0v)
    ones[...] = jnp.full((L,), 1.0, jnp.float32)
    pltpu.async_copy(ones, pert.at[gidx], sem_sc).wait()


def kernel(logits):
  pert, khot, _, _ = _topk_multihot(logits)
  return pert, khot


# speedup vs baseline: 1.0118x; 1.0005x over previous
import functools

import jax
import jax.numpy as jnp
from jax import lax
from jax.experimental import pallas as pl
from jax.experimental.pallas import tpu as pltpu
from jax.experimental.pallas import tpu_sc as plsc

V = 1_000_000
K = 8
L = 16
NTILES = 16
NW = 32

SCNT = 31_264
SCNT_L = V - (NW - 1) * SCNT
SNV = SCNT // L
SNVL = SCNT_L // L
SG = 16
SNSG = SNV // SG
STAILV = SNV - SNSG * SG

CNT = 62_528
CNT_L = V - (NTILES - 1) * CNT
ZB = 15_632
NZ = CNT // ZB
NZL_FULL = CNT_L // ZB
ZREM = CNT_L - NZL_FULL * ZB

CB = 1024
NEG = float("-inf")
BIGI = 2**30

_DNUMS = lax.GatherDimensionNumbers(
    offset_dims=(), collapsed_slice_dims=(0,), start_index_map=(0,))

_mesh = plsc.VectorSubcoreMesh(
    core_axis_name="c", subcore_axis_name="s", num_cores=2)

_rmesh = plsc.VectorSubcoreMesh(
    core_axis_name="rc", subcore_axis_name="rs", num_cores=2)


def _g16(x, idx):
  return lax.gather(x, idx.reshape(L, 1), _DNUMS, (1,),
                    mode=lax.GatherScatterMode.PROMISE_IN_BOUNDS)


def _bf(x, iota, op):
  for k in range(4):
    x = op(x, _g16(x, jnp.bitwise_xor(iota, 1 << k)))
  return x


def _select8(read_v, read_i, nvecs, iota):
  negv = jnp.full((L,), NEG, jnp.float32)
  bigv = jnp.full((L,), BIGI, jnp.int32)
  outv = negv
  outi = bigv
  pvv = jnp.full((L,), float("inf"), jnp.float32)
  piv = jnp.full((L,), -1, jnp.int32)
  for r in range(K):
    def scan(k, carry, pvv=pvv, piv=piv):
      bv, bi = carry
      v = read_v(k)
      ix = read_i(k)
      elig = (v < pvv) | ((v == pvv) & (ix > piv))
      vv = jnp.where(elig, v, negv)
      better = (vv > bv) | ((vv == bv) & (ix < bi))
      bv = jnp.where(better, vv, bv)
      bi = jnp.where(better, ix, bi)
      return bv, bi

    bv, bi = lax.fori_loop(0, nvecs, scan, (negv, bigv))
    mvv = _bf(bv, iota, jnp.maximum)
    miv = _bf(jnp.where(bv == mvv, bi, bigv), iota, jnp.minimum)
    outv = jnp.where(iota == r, mvv, outv)
    outi = jnp.where(iota == r, miv, outi)
    pvv, piv = mvv, miv
  return outv, outi


def _zero_fill(out, zbuf, sem, zbase, wid, last):
  @pl.when(wid < last)
  def _():
    for q in range(NZ):
      pltpu.async_copy(zbuf, out.at[pl.ds(zbase + q * ZB, ZB)], sem)

  @pl.when(wid == last)
  def _():
    for q in range(NZL_FULL):
      pltpu.async_copy(zbuf, out.at[pl.ds(zbase + q * ZB, ZB)], sem)
    zrem_base = zbase + NZL_FULL * ZB
    pltpu.async_copy(zbuf.at[pl.ds(0, ZREM)],
                     out.at[pl.ds(zrem_base, ZREM)], sem)


def _zero_wait(out, zbuf, sem, zbase, wid, last):
  @pl.when(wid < last)
  def _():
    for q in range(NZ):
      pltpu.make_async_copy(
          zbuf, out.at[pl.ds(zbase + q * ZB, ZB)], sem).wait()

  @pl.when(wid == last)
  def _():
    for q in range(NZL_FULL):
      pltpu.make_async_copy(
          zbuf, out.at[pl.ds(zbase + q * ZB, ZB)], sem).wait()
    zrem_base = zbase + NZL_FULL * ZB
    pltpu.make_async_copy(zbuf.at[pl.ds(0, ZREM)],
                          out.at[pl.ds(zrem_base, ZREM)], sem).wait()


@functools.partial(
    pl.kernel,
    out_type=(jax.ShapeDtypeStruct((V,), jnp.float32),
              jax.ShapeDtypeStruct((V,), jnp.float32),
              jax.ShapeDtypeStruct((L,), jnp.float32),
              jax.ShapeDtypeStruct((L,), jnp.int32)),
    mesh=_mesh,
    scratch_types=[
        pltpu.VMEM((SCNT,), jnp.float32),
        pltpu.VMEM((ZB,), jnp.float32),
        pltpu.VMEM((SNSG * L,), jnp.float32),
        pltpu.VMEM((CB,), jnp.float32),
        pltpu.VMEM((CB,), jnp.int32),
        pltpu.VMEM((L,), jnp.float32),
        pltpu.VMEM((L,), jnp.int32),
        pltpu.VMEM_SHARED(((NTILES + 1) * L,), jnp.float32),
        pltpu.VMEM_SHARED(((NTILES + 1) * L,), jnp.int32),
        pltpu.VMEM(((NTILES + 1) * L,), jnp.float32),
        pltpu.VMEM(((NTILES + 1) * L,), jnp.int32),
        pltpu.VMEM((L,), jnp.float32),
        pltpu.VMEM((L,), jnp.int32),
        pltpu.VMEM((L,), jnp.float32),
        pltpu.VMEM((L,), jnp.int32),
        pltpu.SemaphoreType.DMA,
        pltpu.SemaphoreType.DMA,
        pltpu.SemaphoreType.DMA,
        pltpu.SemaphoreType.DMA,
        pltpu.SemaphoreType.REGULAR @ _rmesh,
    ],
)
def _topk_multihot(logits, pert, khot, cvx, cix,
                   xbuf, zbuf, gbuf, cv, ci, tv, ti, sh_v, sh_i, lvb, lib,
                   ones, gidx, rvb, rib, sem_in, sem_in2, sem_z0, sem_sc,
                   rsem):
  cid = lax.axis_index("c")
  wid = lax.axis_index("s")
  w = cid * NTILES + wid
  sbase = w * SCNT
  zbase = wid * CNT
  iota = lax.iota(jnp.int32, L)
  last = NTILES - 1
  wlast = NW - 1
  negv = jnp.full((L,), NEG, jnp.float32)
  bigv = jnp.full((L,), BIGI, jnp.int32)
  zero = jnp.zeros((L,), jnp.float32)

  CH = (SNSG // 2) * SG * L
  pltpu.async_copy(logits.at[pl.ds(sbase, CH)], xbuf.at[pl.ds(0, CH)], sem_in)

  @pl.when(w < wlast)
  def _():
    pltpu.async_copy(logits.at[pl.ds(sbase + CH, SCNT - CH)],
                     xbuf.at[pl.ds(CH, SCNT - CH)], sem_in2)

  @pl.when(w == wlast)
  def _():
    pltpu.async_copy(logits.at[pl.ds(sbase + CH, SCNT_L - CH)],
                     xbuf.at[pl.ds(CH, SCNT_L - CH)], sem_in2)

  def memset_body(i, _):
    o = i * (4 * L)
    zbuf[pl.ds(o, L)] = zero
    zbuf[pl.ds(o + L, L)] = zero
    zbuf[pl.ds(o + 2 * L, L)] = zero
    zbuf[pl.ds(o + 3 * L, L)] = zero
    return 0

  lax.fori_loop(0, ZB // (4 * L), memset_body, 0)
  zbuf[pl.ds(ZB - L, L)] = zero

  @pl.when(cid == 0)
  def _():
    _zero_fill(pert, zbuf, sem_z0, zbase, wid, last)

  @pl.when(cid == 1)
  def _():
    _zero_fill(khot, zbuf, sem_z0, zbase, wid, last)

  def p1(sg, macc):
    o = sg * (SG * L)
    root = jnp.maximum(xbuf[pl.ds(o, L)], xbuf[pl.ds(o + L, L)])
    for j in range(2, SG):
      root = jnp.maximum(root, xbuf[pl.ds(o + j * L, L)])
    gbuf[pl.ds(sg * L, L)] = root
    return jnp.maximum(macc, root)

  pltpu.make_async_copy(logits.at[pl.ds(sbase, CH)],
                        xbuf.at[pl.ds(0, CH)], sem_in).wait()
  macc = lax.fori_loop(0, SNSG // 2, p1, negv)

  @pl.when(w < wlast)
  def _():
    pltpu.make_async_copy(logits.at[pl.ds(sbase + CH, SCNT - CH)],
                          xbuf.at[pl.ds(CH, SCNT - CH)], sem_in2).wait()

  @pl.when(w == wlast)
  def _():
    pltpu.make_async_copy(logits.at[pl.ds(sbase + CH, SCNT_L - CH)],
                          xbuf.at[pl.ds(CH, SCNT_L - CH)], sem_in2).wait()
    for j in range(SNV - SNVL):
      xbuf[pl.ds(SCNT_L + j * L, L)] = negv

  macc = lax.fori_loop(SNSG // 2, SNSG, p1, macc)
  tail_o = SNSG * SG * L
  troot = xbuf[pl.ds(tail_o, L)]
  for j in range(1, STAILV):
    troot = jnp.maximum(troot, xbuf[pl.ds(tail_o + j * L, L)])
  macc = jnp.maximum(macc, troot)

  x = macc
  thrv = negv
  for r in range(K):
    thrv = _bf(x, iota, jnp.maximum)
    if r < K - 1:
      x = jnp.where(x == thrv, negv, x)
  thr_s = thrv[0]

  def vec_update(o, c):
    v = xbuf[pl.ds(o, L)]
    mask = v >= thrv
    vv = jnp.where(mask, v, negv)
    ii = jnp.where(mask, iota + (sbase + o), bigv)
    cv[pl.ds(c, L)] = vv
    ci[pl.ds(c, L)] = ii
    hit = _bf(vv, iota, jnp.maximum)[0] >= thr_s
    return c + jnp.where(hit, jnp.int32(L), jnp.int32(0))

  def p2(sg, c):
    root = gbuf[pl.ds(sg * L, L)]
    has = _bf(root, iota, jnp.maximum)[0] >= thr_s

    def upd(c):
      c = jnp.minimum(c, CB - SG * L)
      o = sg * (SG * L)
      for j in range(SG):
        c = vec_update(o + j * L, c)
      return c

    return lax.cond(has, upd, lambda c: c, c)

  c = lax.fori_loop(0, SNSG, p2, jnp.int32(0))
  c = jnp.minimum(c, CB - STAILV * L)
  for j in range(STAILV):
    c = vec_update(tail_o + j * L, c)

  outv, outi = _select8(lambda k: cv[pl.ds(k * L, L)],
                        lambda k: ci[pl.ds(k * L, L)], c // L, iota)
  tv[...] = outv
  ti[...] = outi
  pltpu.sync_copy(tv, sh_v.at[pl.ds(wid * L, L)])
  pltpu.sync_copy(ti, sh_i.at[pl.ds(wid * L, L)])

  @pl.when(cid == 0)
  def _():
    _zero_wait(pert, zbuf, sem_z0, zbase, wid, last)

  @pl.when(cid == 1)
  def _():
    _zero_wait(khot, zbuf, sem_z0, zbase, wid, last)

  plsc.subcore_barrier()

  target = {"rc": 0, "rs": 0}

  @pl.when(jnp.logical_and(cid == 1, wid == 0))
  def _():
    pltpu.sync_copy(sh_v.at[pl.ds(0, NTILES * L)], lvb.at[pl.ds(0, NTILES * L)])
    pltpu.sync_copy(sh_i.at[pl.ds(0, NTILES * L)], lib.at[pl.ds(0, NTILES * L)])
    v8, i8 = _select8(lambda k: lvb[pl.ds(k * L, L)],
                      lambda k: lib[pl.ds(k * L, L)], NTILES, iota)
    tv[...] = v8
    ti[...] = i8
    pltpu.sync_copy(tv, cvx)
    pltpu.sync_copy(ti, cix)
    pl.semaphore_signal(rsem, 1, device_id=target)

  @pl.when(jnp.logical_and(cid == 0, wid == 0))
  def _():
    d0 = pltpu.async_copy(
        sh_v.at[pl.ds(0, NTILES * L)], lvb.at[pl.ds(0, NTILES * L)], sem_in)
    d1 = pltpu.async_copy(
        sh_i.at[pl.ds(0, NTILES * L)], lib.at[pl.ds(0, NTILES * L)], sem_in)
    pl.semaphore_wait(rsem, 1)
    d2 = pltpu.async_copy(cvx, lvb.at[pl.ds(NTILES * L, L)], sem_z0)
    d3 = pltpu.async_copy(cix, lib.at[pl.ds(NTILES * L, L)], sem_z0)
    d0.wait()
    d1.wait()
    d2.wait()
    d3.wait()
    gv, gi = _select8(lambda k: lvb[pl.ds(k * L, L)],
                      lambda k: lib[pl.ds(k * L, L)], NTILES + 1, iota)
    del gv
    g0v = _g16(gi, jnp.zeros((L,), jnp.int32))
    gidx[...] = jnp.where(iota < K, gi, g0v)
    ones[...] = jnp.full((L,), 1.0, jnp.float32)
    pltpu.async_copy(ones, pert.at[gidx], sem_sc).wait()


def kernel(logits):
  pert, khot, _, _ = _topk_multihot(logits)
  return pert, khot
